# baseline (speedup 1.0000x reference)
import functools

import numpy as np

import jax
import jax.numpy as jnp
from jax import lax
from jax.experimental import pallas as pl
from jax.experimental.pallas import tpu as pltpu
from jax.experimental.pallas import tpu_sc as plsc

_TM = 128
_TB = 512
_META = 256


def _routing_body(nblk, tb, e, tm, nch,
                  x_ref, gw_ref, gb_ref,
                  eid_ref, rank_ref, meta_ref, loss_ref,
                  carry_ref, prob_ref):
    i = pl.program_id(0)

    @pl.when(i == 0)
    def _init():
        carry_ref[...] = jnp.zeros_like(carry_ref)
        prob_ref[...] = jnp.zeros_like(prob_ref)

    xb = x_ref[...]
    scores = lax.dot_general(xb, gw_ref[...], (((1,), (1,)), ((), ())),
                             preferred_element_type=jnp.float32) + gb_ref[...]
    m = jnp.max(scores, axis=1, keepdims=True)
    lane = lax.broadcasted_iota(jnp.int32, scores.shape, 1)
    eid = jnp.min(jnp.where(scores == m, lane, e), axis=1)
    onehot = (lane == eid[:, None]).astype(jnp.float32)
    ex = jnp.exp(scores - m)
    probs = ex / jnp.sum(ex, axis=1, keepdims=True)

    r_i = lax.broadcasted_iota(jnp.int32, (tb, tb), 0)
    c_i = lax.broadcasted_iota(jnp.int32, (tb, tb), 1)
    lt = (c_i < r_i).astype(jnp.float32)
    prev = lax.dot_general(lt, onehot, (((1,), (0,)), ((), ())),
                           preferred_element_type=jnp.float32)
    rank = jnp.sum((prev + carry_ref[...]) * onehot, axis=1)

    eid_ref[...] = eid.astype(jnp.int32).reshape(1, 1, tb)
    rank_ref[...] = rank.astype(jnp.int32).reshape(1, 1, tb)
    carry_ref[...] = carry_ref[...] + jnp.sum(onehot, axis=0, keepdims=True)
    prob_ref[...] = prob_ref[...] + jnp.sum(probs, axis=0, keepdims=True)

    @pl.when(i == nblk - 1)
    def _fin():
        counts = carry_ref[...]
        counts_i = counts.astype(jnp.int32)
        nchunks = (counts_i + (tm - 1)) // tm
        ei = lax.broadcasted_iota(jnp.int32, (e, e), 0)
        fi = lax.broadcasted_iota(jnp.int32, (e, e), 1)
        mle = (ei <= fi).astype(jnp.float32)
        cum = lax.dot_general(nchunks.astype(jnp.float32), mle,
                              (((1,), (0,)), ((), ())),
                              preferred_element_type=jnp.float32
                              ).astype(jnp.int32)
        row_off = (cum - nchunks) * tm
        jj = lax.broadcasted_iota(jnp.int32, (nch, e), 0)
        ince = jnp.sum((jnp.broadcast_to(cum, (nch, e)) <= jj).astype(jnp.int32),
                       axis=1)
        ince = jnp.minimum(ince, e - 1)
        pad = jnp.zeros((_META - e - nch,), jnp.int32)
        meta_ref[...] = jnp.concatenate(
            [row_off.reshape(e), ince, pad]).reshape(1, 1, _META)
        t = nblk * tb
        lossv = (e / (t * t)) * jnp.sum(counts * prob_ref[...])
        loss_ref[...] = jnp.full(loss_ref.shape, lossv, jnp.float32)


def _route(x2, gate_W, gate_b2, *, nblk, tb, e, tm, nch, interpret=False):
    t, d = x2.shape
    return pl.pallas_call(
        functools.partial(_routing_body, nblk, tb, e, tm, nch),
        grid=(nblk,),
        in_specs=[
            pl.BlockSpec((tb, d), lambda i: (i, 0)),
            pl.BlockSpec((e, d), lambda i: (0, 0)),
            pl.BlockSpec((1, e), lambda i: (0, 0)),
        ],
        out_specs=[
            pl.BlockSpec((1, 1, tb), lambda i: (i, 0, 0)),
            pl.BlockSpec((1, 1, tb), lambda i: (i, 0, 0)),
            pl.BlockSpec((1, 1, _META), lambda i: (0, 0, 0)),
            pl.BlockSpec((1, 1, 128), lambda i: (0, 0, 0)),
        ],
        out_shape=[
            jax.ShapeDtypeStruct((nblk, 1, tb), jnp.int32),
            jax.ShapeDtypeStruct((nblk, 1, tb), jnp.int32),
            jax.ShapeDtypeStruct((1, 1, _META), jnp.int32),
            jax.ShapeDtypeStruct((1, 1, 128), jnp.float32),
        ],
        scratch_shapes=[
            pltpu.VMEM((1, e), jnp.float32),
            pltpu.VMEM((1, e), jnp.float32),
        ],
        interpret=interpret,
    )(x2, gate_W, gate_b2)


def _ffn_body(nsplit, inc_ref, x_ref, w1_ref, b1_ref, w2_ref, b2_ref, o_ref):
    p = pl.program_id(1)
    xb = x_ref[...]
    h = lax.dot_general(xb, w1_ref[0], (((1,), (1,)), ((), ())),
                        preferred_element_type=jnp.float32) + b1_ref[0]
    h = h * (lax.erf(h * np.float32(1.0 / np.sqrt(2.0))) + 1.0) * 0.5
    o = lax.dot_general(h, w2_ref[0], (((1,), (1,)), ((), ())),
                        preferred_element_type=jnp.float32)

    @pl.when(p == 0)
    def _first():
        o_ref[...] = o + b2_ref[0]

    @pl.when(p != 0)
    def _rest():
        o_ref[...] = o_ref[...] + o


def _ffn(inc_e, xs, W1, b1, W2, b2, *, nch, tm, nsplit=2, interpret=False):
    e, df, d = W1.shape
    dfb = df // nsplit
    grid_spec = pltpu.PrefetchScalarGridSpec(
        num_scalar_prefetch=1,
        grid=(nch, nsplit),
        in_specs=[
            pl.BlockSpec((tm, d), lambda j, p, inc: (j, 0)),
            pl.BlockSpec((1, dfb, d), lambda j, p, inc: (inc[j], p, 0)),
            pl.BlockSpec((1, 1, dfb), lambda j, p, inc: (inc[j], 0, p)),
            pl.BlockSpec((1, d, dfb), lambda j, p, inc: (inc[j], 0, p)),
            pl.BlockSpec((1, 1, d), lambda j, p, inc: (inc[j], 0, 0)),
        ],
        out_specs=pl.BlockSpec((tm, d), lambda j, p, inc: (j, 0)),
    )
    return pl.pallas_call(
        functools.partial(_ffn_body, nsplit),
        grid_spec=grid_spec,
        out_shape=jax.ShapeDtypeStruct((nch * tm, d), jnp.float32),
        interpret=interpret,
    )(inc_e, xs, W1, b1.reshape(e, 1, df), W2, b2.reshape(e, 1, d))


def _dispatch_body(nc, tpw,
                   x_hbm, eid_hbm, rank_hbm, roff_hbm,
                   xs_hbm, dest_hbm,
                   eid_v, rank_v, roff_v, dest_v, rows_v, sem):
    wid = lax.axis_index("s") * nc + lax.axis_index("c")
    base = wid * tpw
    pltpu.sync_copy(eid_hbm.at[pl.ds(base, tpw)], eid_v)
    pltpu.sync_copy(rank_hbm.at[pl.ds(base, tpw)], rank_v)
    pltpu.sync_copy(roff_hbm, roff_v)
    pltpu.sync_copy(x_hbm.at[pl.ds(base, tpw)], rows_v)
    for i in range(tpw // 16):
        sl = pl.ds(i * 16, 16)
        ro = plsc.load_gather(roff_v, [eid_v[sl]])
        dest_v[sl] = ro + rank_v[sl]
    pltpu.sync_copy(dest_v, dest_hbm.at[pl.ds(base, tpw)])
    pltpu.async_copy(rows_v, xs_hbm.at[dest_v], sem).wait()


def _combine_body(nc, tpw,
                  os_hbm, dest_hbm, out_hbm,
                  dest_v, rows_v, sem):
    wid = lax.axis_index("s") * nc + lax.axis_index("c")
    base = wid * tpw
    pltpu.sync_copy(dest_hbm.at[pl.ds(base, tpw)], dest_v)
    pltpu.async_copy(os_hbm.at[dest_v], rows_v, sem).wait()
    pltpu.sync_copy(rows_v, out_hbm.at[pl.ds(base, tpw)])


def kernel(x, gate_W, gate_b, W1, b1, W2, b2):
    bx, lx, d = x.shape
    e, df, _ = W1.shape
    t = bx * lx
    tm = _TM
    tb = _TB
    nblk = t // tb
    nch = t // tm + e

    x2 = x.reshape(t, d)
    eid3, rank3, meta, loss = _route(
        x2, gate_W, gate_b.reshape(1, e), nblk=nblk, tb=tb, e=e, tm=tm, nch=nch)
    eid = eid3.reshape(t)
    rank = rank3.reshape(t)
    meta1 = meta.reshape(_META)
    row_off = meta1[:e]
    inc_e = meta1[e:e + nch]

    info = plsc.get_sparse_core_info()
    nw = info.num_cores * info.num_subcores
    tpw = t // nw
    mesh = plsc.VectorSubcoreMesh(core_axis_name="c", subcore_axis_name="s")

    dispatch = pl.kernel(
        functools.partial(_dispatch_body, info.num_cores, tpw),
        out_type=[jax.ShapeDtypeStruct((nch * tm, d), jnp.float32),
                  jax.ShapeDtypeStruct((t,), jnp.int32)],
        mesh=mesh,
        compiler_params=pltpu.CompilerParams(needs_layout_passes=False),
        scratch_types=[pltpu.VMEM((tpw,), jnp.int32),
                       pltpu.VMEM((tpw,), jnp.int32),
                       pltpu.VMEM((e,), jnp.int32),
                       pltpu.VMEM((tpw,), jnp.int32),
                       pltpu.VMEM((tpw, d), jnp.float32),
                       pltpu.SemaphoreType.DMA],
    )
    xs, dest = dispatch(x2, eid, rank, row_off)

    os_buf = _ffn(inc_e, xs, W1, b1, W2, b2, nch=nch, tm=tm)

    combine = pl.kernel(
        functools.partial(_combine_body, info.num_cores, tpw),
        out_type=jax.ShapeDtypeStruct((t, d), jnp.float32),
        mesh=mesh,
        scratch_types=[pltpu.VMEM((tpw,), jnp.int32),
                       pltpu.VMEM((tpw, d), jnp.float32),
                       pltpu.SemaphoreType.DMA],
    )
    out2 = combine(os_buf, dest)

    return out2.reshape(bx, lx, d), loss.reshape(-1)[0]


# speedup vs baseline: 1.3829x; 1.3829x over previous
import functools

import numpy as np

import jax
import jax.numpy as jnp
from jax import lax
from jax.experimental import pallas as pl
from jax.experimental.pallas import tpu as pltpu
from jax.experimental.pallas import tpu_sc as plsc

_TM = 128
_TB = 512
_META = 256


def _routing_body(nblk, tb, e, tm, nch,
                  x_ref, gw_ref, gb_ref,
                  eid_ref, rank_ref, meta_ref, loss_ref,
                  carry_ref, prob_ref):
    i = pl.program_id(0)

    @pl.when(i == 0)
    def _init():
        carry_ref[...] = jnp.zeros_like(carry_ref)
        prob_ref[...] = jnp.zeros_like(prob_ref)

    xb = x_ref[...]
    scores = lax.dot_general(xb, gw_ref[...], (((1,), (1,)), ((), ())),
                             preferred_element_type=jnp.float32) + gb_ref[...]
    m = jnp.max(scores, axis=1, keepdims=True)
    lane = lax.broadcasted_iota(jnp.int32, scores.shape, 1)
    eid = jnp.min(jnp.where(scores == m, lane, e), axis=1)
    onehot = (lane == eid[:, None]).astype(jnp.float32)
    ex = jnp.exp(scores - m)
    probs = ex / jnp.sum(ex, axis=1, keepdims=True)

    r_i = lax.broadcasted_iota(jnp.int32, (tb, tb), 0)
    c_i = lax.broadcasted_iota(jnp.int32, (tb, tb), 1)
    lt = (c_i < r_i).astype(jnp.float32)
    prev = lax.dot_general(lt, onehot, (((1,), (0,)), ((), ())),
                           preferred_element_type=jnp.float32)
    rank = jnp.sum((prev + carry_ref[...]) * onehot, axis=1)

    eid_ref[...] = eid.astype(jnp.int32).reshape(1, 1, tb)
    rank_ref[...] = rank.astype(jnp.int32).reshape(1, 1, tb)
    carry_ref[...] = carry_ref[...] + jnp.sum(onehot, axis=0, keepdims=True)
    prob_ref[...] = prob_ref[...] + jnp.sum(probs, axis=0, keepdims=True)

    @pl.when(i == nblk - 1)
    def _fin():
        counts = carry_ref[...]
        counts_i = counts.astype(jnp.int32)
        nchunks = (counts_i + (tm - 1)) // tm
        ei = lax.broadcasted_iota(jnp.int32, (e, e), 0)
        fi = lax.broadcasted_iota(jnp.int32, (e, e), 1)
        mle = (ei <= fi).astype(jnp.float32)
        cum = lax.dot_general(nchunks.astype(jnp.float32), mle,
                              (((1,), (0,)), ((), ())),
                              preferred_element_type=jnp.float32
                              ).astype(jnp.int32)
        row_off = (cum - nchunks) * tm
        nreal = jnp.sum(nchunks)
        jj = lax.broadcasted_iota(jnp.int32, (nch, e), 0)
        jj = jnp.minimum(jj, nreal - 1)
        ince = jnp.sum((jnp.broadcast_to(cum, (nch, e)) <= jj).astype(jnp.int32),
                       axis=1)
        blk = jnp.minimum(lax.broadcasted_iota(jnp.int32, (nch,), 0), nreal - 1)
        meta_ref[...] = jnp.concatenate(
            [row_off.reshape(e), ince, blk]).reshape(1, 1, _META)
        t = nblk * tb
        lossv = (e / (t * t)) * jnp.sum(counts * prob_ref[...])
        loss_ref[...] = jnp.full(loss_ref.shape, lossv, jnp.float32)


def _route(x2, gate_W, gate_b2, *, nblk, tb, e, tm, nch, interpret=False):
    t, d = x2.shape
    return pl.pallas_call(
        functools.partial(_routing_body, nblk, tb, e, tm, nch),
        grid=(nblk,),
        in_specs=[
            pl.BlockSpec((tb, d), lambda i: (i, 0)),
            pl.BlockSpec((e, d), lambda i: (0, 0)),
            pl.BlockSpec((1, e), lambda i: (0, 0)),
        ],
        out_specs=[
            pl.BlockSpec((1, 1, tb), lambda i: (i, 0, 0)),
            pl.BlockSpec((1, 1, tb), lambda i: (i, 0, 0)),
            pl.BlockSpec((1, 1, _META), lambda i: (0, 0, 0)),
            pl.BlockSpec((1, 1, 128), lambda i: (0, 0, 0)),
        ],
        out_shape=[
            jax.ShapeDtypeStruct((nblk, 1, tb), jnp.int32),
            jax.ShapeDtypeStruct((nblk, 1, tb), jnp.int32),
            jax.ShapeDtypeStruct((1, 1, _META), jnp.int32),
            jax.ShapeDtypeStruct((1, 1, 128), jnp.float32),
        ],
        scratch_shapes=[
            pltpu.VMEM((1, e), jnp.float32),
            pltpu.VMEM((1, e), jnp.float32),
        ],
        interpret=interpret,
    )(x2, gate_W, gate_b2)


def _ffn_body(inc_ref, blk_ref, x_ref, w1_ref, b1_ref, w2_ref, b2_ref, o_ref):
    xb = x_ref[...]
    h = lax.dot_general(xb, w1_ref[0], (((1,), (1,)), ((), ())),
                        preferred_element_type=jnp.float32) + b1_ref[0]
    h = h * (lax.erf(h * np.float32(1.0 / np.sqrt(2.0))) + 1.0) * 0.5
    o = lax.dot_general(h, w2_ref[0], (((1,), (1,)), ((), ())),
                        preferred_element_type=jnp.float32) + b2_ref[0]
    o_ref[...] = o


def _ffn(inc_e, blk, xs, W1, b1, W2, b2, *, nch, tm, interpret=False):
    e, df, d = W1.shape
    grid_spec = pltpu.PrefetchScalarGridSpec(
        num_scalar_prefetch=2,
        grid=(nch,),
        in_specs=[
            pl.BlockSpec((tm, d), lambda j, inc, blk: (blk[j], 0)),
            pl.BlockSpec((1, df, d), lambda j, inc, blk: (inc[j], 0, 0)),
            pl.BlockSpec((1, 1, df), lambda j, inc, blk: (inc[j], 0, 0)),
            pl.BlockSpec((1, d, df), lambda j, inc, blk: (inc[j], 0, 0)),
            pl.BlockSpec((1, 1, d), lambda j, inc, blk: (inc[j], 0, 0)),
        ],
        out_specs=pl.BlockSpec((tm, d), lambda j, inc, blk: (blk[j], 0)),
    )
    return pl.pallas_call(
        _ffn_body,
        grid_spec=grid_spec,
        out_shape=jax.ShapeDtypeStruct((nch * tm, d), jnp.float32),
        interpret=interpret,
    )(inc_e, blk, xs, W1, b1.reshape(e, 1, df), W2, b2.reshape(e, 1, d))


def _dispatch_body(nc, tpw,
                   x_hbm, eid_hbm, rank_hbm, roff_hbm,
                   xs_hbm, dest_hbm,
                   eid_v, rank_v, roff_v, dest_v, rows_v, sem):
    wid = lax.axis_index("s") * nc + lax.axis_index("c")
    base = wid * tpw
    pltpu.sync_copy(eid_hbm.at[pl.ds(base, tpw)], eid_v)
    pltpu.sync_copy(rank_hbm.at[pl.ds(base, tpw)], rank_v)
    pltpu.sync_copy(roff_hbm, roff_v)
    pltpu.sync_copy(x_hbm.at[pl.ds(base, tpw)], rows_v)
    for i in range(tpw // 16):
        sl = pl.ds(i * 16, 16)
        ro = plsc.load_gather(roff_v, [eid_v[sl]])
        dest_v[sl] = ro + rank_v[sl]
    pltpu.sync_copy(dest_v, dest_hbm.at[pl.ds(base, tpw)])
    pltpu.async_copy(rows_v, xs_hbm.at[dest_v], sem).wait()


def _combine_body(nc, tpw,
                  os_hbm, dest_hbm, out_hbm,
                  dest_v, rows_v, sem):
    wid = lax.axis_index("s") * nc + lax.axis_index("c")
    base = wid * tpw
    pltpu.sync_copy(dest_hbm.at[pl.ds(base, tpw)], dest_v)
    pltpu.async_copy(os_hbm.at[dest_v], rows_v, sem).wait()
    pltpu.sync_copy(rows_v, out_hbm.at[pl.ds(base, tpw)])


def kernel(x, gate_W, gate_b, W1, b1, W2, b2):
    bx, lx, d = x.shape
    e, df, _ = W1.shape
    t = bx * lx
    tm = _TM
    tb = _TB
    nblk = t // tb
    nch = t // tm + e

    x2 = x.reshape(t, d)
    eid3, rank3, meta, loss = _route(
        x2, gate_W, gate_b.reshape(1, e), nblk=nblk, tb=tb, e=e, tm=tm, nch=nch)
    eid = eid3.reshape(t)
    rank = rank3.reshape(t)
    meta1 = meta.reshape(_META)
    row_off = meta1[:e]
    inc_e = meta1[e:e + nch]
    blk = meta1[e + nch:e + 2 * nch]

    info = plsc.get_sparse_core_info()
    nw = info.num_cores * info.num_subcores
    tpw = t // nw
    mesh = plsc.VectorSubcoreMesh(core_axis_name="c", subcore_axis_name="s")

    dispatch = pl.kernel(
        functools.partial(_dispatch_body, info.num_cores, tpw),
        out_type=[jax.ShapeDtypeStruct((nch * tm, d), jnp.float32),
                  jax.ShapeDtypeStruct((t,), jnp.int32)],
        mesh=mesh,
        compiler_params=pltpu.CompilerParams(needs_layout_passes=False),
        scratch_types=[pltpu.VMEM((tpw,), jnp.int32),
                       pltpu.VMEM((tpw,), jnp.int32),
                       pltpu.VMEM((e,), jnp.int32),
                       pltpu.VMEM((tpw,), jnp.int32),
                       pltpu.VMEM((tpw, d), jnp.float32),
                       pltpu.SemaphoreType.DMA],
    )
    xs, dest = dispatch(x2, eid, rank, row_off)

    os_buf = _ffn(inc_e, blk, xs, W1, b1, W2, b2, nch=nch, tm=tm)

    combine = pl.kernel(
        functools.partial(_combine_body, info.num_cores, tpw),
        out_type=jax.ShapeDtypeStruct((t, d), jnp.float32),
        mesh=mesh,
        scratch_types=[pltpu.VMEM((tpw,), jnp.int32),
                       pltpu.VMEM((tpw, d), jnp.float32),
                       pltpu.SemaphoreType.DMA],
    )
    out2 = combine(os_buf, dest)

    return out2.reshape(bx, lx, d), loss.reshape(-1)[0]


# speedup vs baseline: 1.6576x; 1.1986x over previous
import functools

import numpy as np

import jax
import jax.numpy as jnp
from jax import lax
from jax.experimental import pallas as pl
from jax.experimental.pallas import tpu as pltpu
from jax.experimental.pallas import tpu_sc as plsc

_TM = 128
_TB = 512
_META = 256


def _routing_body(nblk, tb, e, tm, nch,
                  x_ref, gw_ref, gb_ref,
                  eid_ref, rank_ref, meta_ref, loss_ref,
                  carry_ref, prob_ref):
    i = pl.program_id(0)

    @pl.when(i == 0)
    def _init():
        carry_ref[...] = jnp.zeros_like(carry_ref)
        prob_ref[...] = jnp.zeros_like(prob_ref)

    xb = x_ref[...]
    scores = lax.dot_general(xb, gw_ref[...], (((1,), (1,)), ((), ())),
                             preferred_element_type=jnp.float32) + gb_ref[...]
    m = jnp.max(scores, axis=1, keepdims=True)
    lane = lax.broadcasted_iota(jnp.int32, scores.shape, 1)
    eid = jnp.min(jnp.where(scores == m, lane, e), axis=1)
    onehot = (lane == eid[:, None]).astype(jnp.float32)
    ex = jnp.exp(scores - m)
    probs = ex / jnp.sum(ex, axis=1, keepdims=True)

    r_i = lax.broadcasted_iota(jnp.int32, (tb, tb), 0)
    c_i = lax.broadcasted_iota(jnp.int32, (tb, tb), 1)
    lt = (c_i < r_i).astype(jnp.float32)
    prev = lax.dot_general(lt, onehot, (((1,), (0,)), ((), ())),
                           preferred_element_type=jnp.float32)
    rank = jnp.sum((prev + carry_ref[...]) * onehot, axis=1)

    eid_ref[...] = eid.astype(jnp.int32).reshape(1, 1, tb)
    rank_ref[...] = rank.astype(jnp.int32).reshape(1, 1, tb)
    carry_ref[...] = carry_ref[...] + jnp.sum(onehot, axis=0, keepdims=True)
    prob_ref[...] = prob_ref[...] + jnp.sum(probs, axis=0, keepdims=True)

    @pl.when(i == nblk - 1)
    def _fin():
        counts = carry_ref[...]
        counts_i = counts.astype(jnp.int32)
        nchunks = (counts_i + (tm - 1)) // tm
        ei = lax.broadcasted_iota(jnp.int32, (e, e), 0)
        fi = lax.broadcasted_iota(jnp.int32, (e, e), 1)
        mle = (ei <= fi).astype(jnp.float32)
        cum = lax.dot_general(nchunks.astype(jnp.float32), mle,
                              (((1,), (0,)), ((), ())),
                              preferred_element_type=jnp.float32
                              ).astype(jnp.int32)
        row_off = (cum - nchunks) * tm
        nreal = jnp.sum(nchunks)
        jj = lax.broadcasted_iota(jnp.int32, (nch, e), 0)
        jj = jnp.minimum(jj, nreal - 1)
        ince = jnp.sum((jnp.broadcast_to(cum, (nch, e)) <= jj).astype(jnp.int32),
                       axis=1)
        blk = jnp.minimum(lax.broadcasted_iota(jnp.int32, (nch,), 0), nreal - 1)
        meta_ref[...] = jnp.concatenate(
            [row_off.reshape(e), ince, blk]).reshape(1, 1, _META)
        t = nblk * tb
        lossv = (e / (t * t)) * jnp.sum(counts * prob_ref[...])
        loss_ref[...] = jnp.full(loss_ref.shape, lossv, jnp.float32)


def _route(x2, gate_W, gate_b2, *, nblk, tb, e, tm, nch, interpret=False):
    t, d = x2.shape
    return pl.pallas_call(
        functools.partial(_routing_body, nblk, tb, e, tm, nch),
        grid=(nblk,),
        in_specs=[
            pl.BlockSpec((tb, d), lambda i: (i, 0)),
            pl.BlockSpec((e, d), lambda i: (0, 0)),
            pl.BlockSpec((1, e), lambda i: (0, 0)),
        ],
        out_specs=[
            pl.BlockSpec((1, 1, tb), lambda i: (i, 0, 0)),
            pl.BlockSpec((1, 1, tb), lambda i: (i, 0, 0)),
            pl.BlockSpec((1, 1, _META), lambda i: (0, 0, 0)),
            pl.BlockSpec((1, 1, 128), lambda i: (0, 0, 0)),
        ],
        out_shape=[
            jax.ShapeDtypeStruct((nblk, 1, tb), jnp.int32),
            jax.ShapeDtypeStruct((nblk, 1, tb), jnp.int32),
            jax.ShapeDtypeStruct((1, 1, _META), jnp.int32),
            jax.ShapeDtypeStruct((1, 1, 128), jnp.float32),
        ],
        scratch_shapes=[
            pltpu.VMEM((1, e), jnp.float32),
            pltpu.VMEM((1, e), jnp.float32),
        ],
        interpret=interpret,
    )(x2, gate_W, gate_b2)


def _ffn_body(inc_ref, blk_ref, x_ref, w1_ref, b1_ref, w2_ref, b2_ref, o_ref):
    j = pl.program_id(0)

    @pl.when(blk_ref[j] == j)
    def _real():
        xb = x_ref[...]
        h = lax.dot_general(xb, w1_ref[0], (((1,), (1,)), ((), ())),
                            preferred_element_type=jnp.float32) + b1_ref[0]
        h = h * (lax.erf(h * np.float32(1.0 / np.sqrt(2.0))) + 1.0) * 0.5
        o = lax.dot_general(h, w2_ref[0], (((1,), (1,)), ((), ())),
                            preferred_element_type=jnp.float32) + b2_ref[0]
        o_ref[...] = o


def _ffn(inc_e, blk, xs, W1, b1, W2, b2, *, nch, tm, interpret=False):
    e, df, d = W1.shape
    grid_spec = pltpu.PrefetchScalarGridSpec(
        num_scalar_prefetch=2,
        grid=(nch,),
        in_specs=[
            pl.BlockSpec((tm, d), lambda j, inc, blk: (blk[j], 0)),
            pl.BlockSpec((1, df, d), lambda j, inc, blk: (inc[j], 0, 0)),
            pl.BlockSpec((1, 1, df), lambda j, inc, blk: (inc[j], 0, 0)),
            pl.BlockSpec((1, d, df), lambda j, inc, blk: (inc[j], 0, 0)),
            pl.BlockSpec((1, 1, d), lambda j, inc, blk: (inc[j], 0, 0)),
        ],
        out_specs=pl.BlockSpec((tm, d), lambda j, inc, blk: (blk[j], 0)),
    )
    return pl.pallas_call(
        _ffn_body,
        grid_spec=grid_spec,
        out_shape=jax.ShapeDtypeStruct((nch * tm, d), jnp.float32),
        interpret=interpret,
    )(inc_e, blk, xs, W1, b1.reshape(e, 1, df), W2, b2.reshape(e, 1, d))


def _dispatch_body(nc, tpw,
                   x_hbm, eid_hbm, rank_hbm, roff_hbm,
                   xs_hbm, dest_hbm,
                   eid_v, rank_v, roff_v, dest_v, rows_v, sem):
    wid = lax.axis_index("s") * nc + lax.axis_index("c")
    base = wid * tpw
    pltpu.sync_copy(eid_hbm.at[pl.ds(base, tpw)], eid_v)
    pltpu.sync_copy(rank_hbm.at[pl.ds(base, tpw)], rank_v)
    pltpu.sync_copy(roff_hbm, roff_v)
    pltpu.sync_copy(x_hbm.at[pl.ds(base, tpw)], rows_v)
    for i in range(tpw // 16):
        sl = pl.ds(i * 16, 16)
        ro = plsc.load_gather(roff_v, [eid_v[sl]])
        dest_v[sl] = ro + rank_v[sl]
    pltpu.sync_copy(dest_v, dest_hbm.at[pl.ds(base, tpw)])
    pltpu.async_copy(rows_v, xs_hbm.at[dest_v], sem).wait()


def _combine_body(nc, tpw,
                  os_hbm, dest_hbm, out_hbm,
                  dest_v, rows_v, sem):
    wid = lax.axis_index("s") * nc + lax.axis_index("c")
    base = wid * tpw
    pltpu.sync_copy(dest_hbm.at[pl.ds(base, tpw)], dest_v)
    pltpu.async_copy(os_hbm.at[dest_v], rows_v, sem).wait()
    pltpu.sync_copy(rows_v, out_hbm.at[pl.ds(base, tpw)])


def kernel(x, gate_W, gate_b, W1, b1, W2, b2):
    bx, lx, d = x.shape
    e, df, _ = W1.shape
    t = bx * lx
    tm = _TM
    tb = _TB
    nblk = t // tb
    nch = t // tm + e

    x2 = x.reshape(t, d)
    eid3, rank3, meta, loss = _route(
        x2, gate_W, gate_b.reshape(1, e), nblk=nblk, tb=tb, e=e, tm=tm, nch=nch)
    eid = eid3.reshape(t)
    rank = rank3.reshape(t)
    meta1 = meta.reshape(_META)
    row_off = meta1[:e]
    inc_e = meta1[e:e + nch]
    blk = meta1[e + nch:e + 2 * nch]

    info = plsc.get_sparse_core_info()
    nw = info.num_cores * info.num_subcores
    tpw = t // nw
    mesh = plsc.VectorSubcoreMesh(core_axis_name="c", subcore_axis_name="s")

    dispatch = pl.kernel(
        functools.partial(_dispatch_body, info.num_cores, tpw),
        out_type=[jax.ShapeDtypeStruct((nch * tm, d), jnp.float32),
                  jax.ShapeDtypeStruct((t,), jnp.int32)],
        mesh=mesh,
        compiler_params=pltpu.CompilerParams(needs_layout_passes=False),
        scratch_types=[pltpu.VMEM((tpw,), jnp.int32),
                       pltpu.VMEM((tpw,), jnp.int32),
                       pltpu.VMEM((e,), jnp.int32),
                       pltpu.VMEM((tpw,), jnp.int32),
                       pltpu.VMEM((tpw, d), jnp.float32),
                       pltpu.SemaphoreType.DMA],
    )
    xs, dest = dispatch(x2, eid, rank, row_off)

    os_buf = _ffn(inc_e, blk, xs, W1, b1, W2, b2, nch=nch, tm=tm)

    combine = pl.kernel(
        functools.partial(_combine_body, info.num_cores, tpw),
        out_type=jax.ShapeDtypeStruct((t, d), jnp.float32),
        mesh=mesh,
        scratch_types=[pltpu.VMEM((tpw,), jnp.int32),
                       pltpu.VMEM((tpw, d), jnp.float32),
                       pltpu.SemaphoreType.DMA],
    )
    out2 = combine(os_buf, dest)

    return out2.reshape(bx, lx, d), loss.reshape(-1)[0]
